# Initial kernel scaffold; baseline (speedup 1.0000x reference)
import functools

import jax
import jax.numpy as jnp
from jax import lax
from jax.experimental import pallas as pl
from jax.experimental.pallas import tpu as pltpu
from jax.experimental.pallas import tpu_sc as plsc

N_NODES = 10000
N_EDGES = 320000
IN_DIM = 128
H_DIM = 32

N_TILES = 32
E_PAD = 327680
IDX_ROWS = E_PAD // 128
ROWS_PER_TILE = IDX_ROWS // N_TILES
CHUNK_ROWS = 8
N_CHUNKS = ROWS_PER_TILE // CHUNK_ROWS
N_PAD = 10240
SLICE = N_PAD // 16
N_BLK = 79
N_TC = N_BLK * 128

_mesh = plsc.VectorSubcoreMesh(core_axis_name="c", subcore_axis_name="s")


@functools.partial(
    pl.kernel,
    out_type=jax.ShapeDtypeStruct((2, N_PAD), jnp.float32),
    mesh=_mesh,
    scratch_types=[
        pltpu.VMEM((CHUNK_ROWS, 128), jnp.int32),
        pltpu.VMEM((SLICE,), jnp.float32),
        pltpu.VMEM((128,), jnp.float32),
        pltpu.VMEM_SHARED((N_PAD,), jnp.float32),
    ],
)
def _sc_degree(dst_hbm, zeros_hbm, ones_hbm, out_hbm, dst_v, stage_v, ones_v, deg_sh):
    c = lax.axis_index("c")
    s = lax.axis_index("s")
    wid = c * 16 + s
    pltpu.sync_copy(zeros_hbm, stage_v)
    pltpu.sync_copy(stage_v, deg_sh.at[pl.ds(s * SLICE, SLICE)])
    pltpu.sync_copy(ones_hbm, ones_v)
    plsc.subcore_barrier()
    base = wid * ROWS_PER_TILE

    def body(i, carry):
        pltpu.sync_copy(dst_hbm.at[pl.ds(base + i * CHUNK_ROWS, CHUNK_ROWS)], dst_v)
        for j in range(CHUNK_ROWS):
            pltpu.sync_copy(ones_v, deg_sh.at[dst_v.at[j]], add=True)
        return carry

    lax.fori_loop(0, N_CHUNKS, body, 0)
    plsc.subcore_barrier()
    pltpu.sync_copy(deg_sh.at[pl.ds(s * SLICE, SLICE)],
                    out_hbm.at[c, pl.ds(s * SLICE, SLICE)])


@functools.partial(
    pl.kernel,
    out_type=jax.ShapeDtypeStruct((2, N_PAD, H_DIM), jnp.float32),
    mesh=_mesh,
    scratch_types=[
        pltpu.VMEM((CHUNK_ROWS, 128), jnp.int32),
        pltpu.VMEM((CHUNK_ROWS, 128), jnp.int32),
        pltpu.VMEM((CHUNK_ROWS * 128, H_DIM), jnp.float32),
        pltpu.VMEM_SHARED((N_PAD, H_DIM), jnp.float32),
        pltpu.SemaphoreType.DMA,
    ],
)
def _sc_aggregate(h_hbm, src_hbm, dst_hbm, zeros_hbm, out_hbm,
                  src_v, dst_v, rows_v, acc_sh, sem):
    c = lax.axis_index("c")
    s = lax.axis_index("s")
    wid = c * 16 + s
    pltpu.sync_copy(zeros_hbm, rows_v.at[pl.ds(0, SLICE)])
    pltpu.sync_copy(rows_v.at[pl.ds(0, SLICE)], acc_sh.at[pl.ds(s * SLICE, SLICE)])
    plsc.subcore_barrier()
    base = wid * ROWS_PER_TILE

    def body(i, carry):
        r0 = base + i * CHUNK_ROWS
        pltpu.sync_copy(src_hbm.at[pl.ds(r0, CHUNK_ROWS)], src_v)
        pltpu.sync_copy(dst_hbm.at[pl.ds(r0, CHUNK_ROWS)], dst_v)
        for j in range(CHUNK_ROWS):
            pltpu.async_copy(h_hbm.at[src_v.at[j]],
                             rows_v.at[pl.ds(j * 128, 128)], sem).wait()
        for j in range(CHUNK_ROWS):
            pltpu.sync_copy(rows_v.at[pl.ds(j * 128, 128)],
                            acc_sh.at[dst_v.at[j]], add=True)
        return carry

    lax.fori_loop(0, N_CHUNKS, body, 0)
    plsc.subcore_barrier()
    pltpu.sync_copy(acc_sh.at[pl.ds(s * SLICE, SLICE)],
                    out_hbm.at[c, pl.ds(s * SLICE, SLICE)])


def _fuse1_body(degp_ref, x_ref, w1_ref, dis_ref, h_ref):
    deg = degp_ref[0] + degp_ref[1] + 1.0
    dis = lax.rsqrt(deg)
    dis_ref[...] = dis
    h = jnp.dot(x_ref[...], w1_ref[...], preferred_element_type=jnp.float32)
    h_ref[...] = h * dis


def _fuse2_body(p_ref, h1p_ref, dis_ref, b1_ref, w2_ref, out_ref):
    dis = dis_ref[...]
    acc = p_ref[0] + p_ref[1] + h1p_ref[...]
    h1 = jnp.maximum(acc * dis + b1_ref[...], 0.0)
    out_ref[...] = jnp.dot(h1, w2_ref[...], preferred_element_type=jnp.float32) * dis


def _fuse3_body(p_ref, h2p_ref, dis_ref, b2_ref, wh_ref, bh_ref, out_ref):
    acc = p_ref[0] + p_ref[1] + h2p_ref[...]
    h2 = jnp.maximum(acc * dis_ref[...] + b2_ref[...], 0.0)
    out_ref[...] = jnp.dot(h2, wh_ref[...], preferred_element_type=jnp.float32) + bh_ref[...]


def _tc_fuse1(degp, x_pad, w1):
    return pl.pallas_call(
        _fuse1_body,
        grid=(N_BLK,),
        in_specs=[
            pl.BlockSpec((2, 128, 1), lambda i: (0, i, 0)),
            pl.BlockSpec((128, IN_DIM), lambda i: (i, 0)),
            pl.BlockSpec((IN_DIM, H_DIM), lambda i: (0, 0)),
        ],
        out_specs=[
            pl.BlockSpec((128, 1), lambda i: (i, 0)),
            pl.BlockSpec((128, H_DIM), lambda i: (i, 0)),
        ],
        out_shape=[
            jax.ShapeDtypeStruct((N_TC, 1), jnp.float32),
            jax.ShapeDtypeStruct((N_TC, H_DIM), jnp.float32),
        ],
    )(degp, x_pad, w1)


def _tc_fuse2(p, h1p, dis, b1, w2):
    return pl.pallas_call(
        _fuse2_body,
        grid=(N_BLK,),
        in_specs=[
            pl.BlockSpec((2, 128, H_DIM), lambda i: (0, i, 0)),
            pl.BlockSpec((128, H_DIM), lambda i: (i, 0)),
            pl.BlockSpec((128, 1), lambda i: (i, 0)),
            pl.BlockSpec((1, H_DIM), lambda i: (0, 0)),
            pl.BlockSpec((H_DIM, H_DIM), lambda i: (0, 0)),
        ],
        out_specs=pl.BlockSpec((128, H_DIM), lambda i: (i, 0)),
        out_shape=jax.ShapeDtypeStruct((N_TC, H_DIM), jnp.float32),
    )(p, h1p, dis, b1, w2)


def _tc_fuse3(p, h2p, dis, b2, wh_pad, bh_pad):
    return pl.pallas_call(
        _fuse3_body,
        grid=(N_BLK,),
        in_specs=[
            pl.BlockSpec((2, 128, H_DIM), lambda i: (0, i, 0)),
            pl.BlockSpec((128, H_DIM), lambda i: (i, 0)),
            pl.BlockSpec((128, 1), lambda i: (i, 0)),
            pl.BlockSpec((1, H_DIM), lambda i: (0, 0)),
            pl.BlockSpec((H_DIM, 128), lambda i: (0, 0)),
            pl.BlockSpec((1, 128), lambda i: (0, 0)),
        ],
        out_specs=pl.BlockSpec((128, 128), lambda i: (i, 0)),
        out_shape=jax.ShapeDtypeStruct((N_TC, 128), jnp.float32),
    )(p, h2p, dis, b2, wh_pad, bh_pad)


def kernel(x, edge_index, W1, b1, W2, b2, Wh, bh):
    src = edge_index[0].astype(jnp.int32)
    dst = edge_index[1].astype(jnp.int32)
    src = jnp.concatenate([src, jnp.zeros((E_PAD - N_EDGES,), jnp.int32)])
    dst = jnp.concatenate(
        [dst, jnp.full((E_PAD - N_EDGES,), N_NODES, jnp.int32)])
    src2d = src.reshape(IDX_ROWS, 128)
    dst2d = dst.reshape(IDX_ROWS, 128)

    zeros_w = jnp.zeros((SLICE, H_DIM), jnp.float32)
    zeros_1 = jnp.zeros((SLICE,), jnp.float32)
    ones_1 = jnp.ones((128,), jnp.float32)

    x_pad = jnp.pad(x, ((0, N_TC - N_NODES), (0, 0)))
    wh_pad = jnp.pad(Wh, ((0, 0), (0, 128 - Wh.shape[1])))
    bh_pad = jnp.pad(bh, (0, 128 - bh.shape[0])).reshape(1, 128)
    b1r = b1.reshape(1, H_DIM)
    b2r = b2.reshape(1, H_DIM)

    degp = _sc_degree(dst2d, zeros_1, ones_1)
    degp = degp[:, :N_TC].reshape(2, N_TC, 1)
    dis, h1p = _tc_fuse1(degp, x_pad, W1)

    p1 = _sc_aggregate(h1p, src2d, dst2d, zeros_w)
    h2p = _tc_fuse2(p1[:, :N_TC], h1p, dis, b1r, W2)

    p2 = _sc_aggregate(h2p, src2d, dst2d, zeros_w)
    logits_pad = _tc_fuse3(p2[:, :N_TC], h2p, dis, b2r, wh_pad, bh_pad)
    return logits_pad[:N_NODES, :Wh.shape[1]]


# speedup vs baseline: 15.7799x; 15.7799x over previous
import functools

import jax
import jax.numpy as jnp
from jax import lax
from jax.experimental import pallas as pl
from jax.experimental.pallas import tpu as pltpu
from jax.experimental.pallas import tpu_sc as plsc

N_NODES = 10000
N_EDGES = 320000
IN_DIM = 128
H_DIM = 32

N_TILES = 32
E_PAD = 327680
IDX_ROWS = E_PAD // 128
ROWS_PER_TILE = IDX_ROWS // N_TILES
CHUNK_ROWS = 8
N_CHUNKS = ROWS_PER_TILE // CHUNK_ROWS
N_PAD = 10240
SLICE = N_PAD // 16
N_BLK = 79
N_TC = N_BLK * 128

_mesh = plsc.VectorSubcoreMesh(core_axis_name="c", subcore_axis_name="s")


@functools.partial(
    pl.kernel,
    out_type=jax.ShapeDtypeStruct((2, N_PAD), jnp.float32),
    mesh=_mesh,
    scratch_types=[
        pltpu.VMEM((CHUNK_ROWS, 128), jnp.int32),
        pltpu.VMEM((SLICE,), jnp.float32),
        pltpu.VMEM((128,), jnp.float32),
        pltpu.VMEM_SHARED((N_PAD,), jnp.float32),
    ],
)
def _sc_degree(dst_hbm, zeros_hbm, ones_hbm, out_hbm, dst_v, stage_v, ones_v, deg_sh):
    c = lax.axis_index("c")
    s = lax.axis_index("s")
    wid = c * 16 + s
    pltpu.sync_copy(zeros_hbm, stage_v)
    pltpu.sync_copy(stage_v, deg_sh.at[pl.ds(s * SLICE, SLICE)])
    pltpu.sync_copy(ones_hbm, ones_v)
    plsc.subcore_barrier()
    base = wid * ROWS_PER_TILE

    def body(i, carry):
        pltpu.sync_copy(dst_hbm.at[pl.ds(base + i * CHUNK_ROWS, CHUNK_ROWS)], dst_v)
        for j in range(CHUNK_ROWS):
            pltpu.sync_copy(ones_v, deg_sh.at[dst_v.at[j]], add=True)
        return carry

    lax.fori_loop(0, N_CHUNKS, body, 0)
    plsc.subcore_barrier()
    pltpu.sync_copy(deg_sh.at[pl.ds(s * SLICE, SLICE)],
                    out_hbm.at[c, pl.ds(s * SLICE, SLICE)])


@functools.partial(
    pl.kernel,
    out_type=jax.ShapeDtypeStruct((2, N_PAD, H_DIM), jnp.float32),
    mesh=_mesh,
    scratch_types=[
        pltpu.VMEM((CHUNK_ROWS, 128), jnp.int32),
        pltpu.VMEM((CHUNK_ROWS, 128), jnp.int32),
        pltpu.VMEM((CHUNK_ROWS * 128, H_DIM), jnp.float32),
        pltpu.VMEM_SHARED((N_PAD, H_DIM), jnp.float32),
        pltpu.SemaphoreType.DMA,
    ],
    compiler_params=pltpu.CompilerParams(use_tc_tiling_on_sc=False),
)
def _sc_aggregate(h_hbm, src_hbm, dst_hbm, zeros_hbm, out_hbm,
                  src_v, dst_v, rows_v, acc_sh, sem):
    c = lax.axis_index("c")
    s = lax.axis_index("s")
    wid = c * 16 + s
    pltpu.sync_copy(zeros_hbm, rows_v.at[pl.ds(0, SLICE)])
    pltpu.sync_copy(rows_v.at[pl.ds(0, SLICE)], acc_sh.at[pl.ds(s * SLICE, SLICE)])
    plsc.subcore_barrier()
    base = wid * ROWS_PER_TILE

    def body(i, carry):
        r0 = base + i * CHUNK_ROWS
        pltpu.sync_copy(src_hbm.at[pl.ds(r0, CHUNK_ROWS)], src_v)
        pltpu.sync_copy(dst_hbm.at[pl.ds(r0, CHUNK_ROWS)], dst_v)
        for j in range(CHUNK_ROWS):
            pltpu.async_copy(h_hbm.at[src_v.at[j]],
                             rows_v.at[pl.ds(j * 128, 128)], sem).wait()
        for j in range(CHUNK_ROWS):
            pltpu.sync_copy(rows_v.at[pl.ds(j * 128, 128)],
                            acc_sh.at[dst_v.at[j]], add=True)
        return carry

    lax.fori_loop(0, N_CHUNKS, body, 0)
    plsc.subcore_barrier()
    pltpu.sync_copy(acc_sh.at[pl.ds(s * SLICE, SLICE)],
                    out_hbm.at[c, pl.ds(s * SLICE, SLICE)])


def _fuse1_body(degp_ref, x_ref, w1_ref, dis_ref, h_ref):
    deg = degp_ref[0] + degp_ref[1] + 1.0
    dis = lax.rsqrt(deg)
    dis_ref[...] = dis
    h = jnp.dot(x_ref[...], w1_ref[...], preferred_element_type=jnp.float32)
    h_ref[...] = h * dis


def _fuse2_body(p_ref, h1p_ref, dis_ref, b1_ref, w2_ref, out_ref):
    dis = dis_ref[...]
    acc = p_ref[0] + p_ref[1] + h1p_ref[...]
    h1 = jnp.maximum(acc * dis + b1_ref[...], 0.0)
    out_ref[...] = jnp.dot(h1, w2_ref[...], preferred_element_type=jnp.float32) * dis


def _fuse3_body(p_ref, h2p_ref, dis_ref, b2_ref, wh_ref, bh_ref, out_ref):
    acc = p_ref[0] + p_ref[1] + h2p_ref[...]
    h2 = jnp.maximum(acc * dis_ref[...] + b2_ref[...], 0.0)
    out_ref[...] = jnp.dot(h2, wh_ref[...], preferred_element_type=jnp.float32) + bh_ref[...]


def _tc_fuse1(degp, x_pad, w1):
    return pl.pallas_call(
        _fuse1_body,
        grid=(N_BLK,),
        in_specs=[
            pl.BlockSpec((2, 128, 1), lambda i: (0, i, 0)),
            pl.BlockSpec((128, IN_DIM), lambda i: (i, 0)),
            pl.BlockSpec((IN_DIM, H_DIM), lambda i: (0, 0)),
        ],
        out_specs=[
            pl.BlockSpec((128, 1), lambda i: (i, 0)),
            pl.BlockSpec((128, H_DIM), lambda i: (i, 0)),
        ],
        out_shape=[
            jax.ShapeDtypeStruct((N_TC, 1), jnp.float32),
            jax.ShapeDtypeStruct((N_TC, H_DIM), jnp.float32),
        ],
    )(degp, x_pad, w1)


def _tc_fuse2(p, h1p, dis, b1, w2):
    return pl.pallas_call(
        _fuse2_body,
        grid=(N_BLK,),
        in_specs=[
            pl.BlockSpec((2, 128, H_DIM), lambda i: (0, i, 0)),
            pl.BlockSpec((128, H_DIM), lambda i: (i, 0)),
            pl.BlockSpec((128, 1), lambda i: (i, 0)),
            pl.BlockSpec((1, H_DIM), lambda i: (0, 0)),
            pl.BlockSpec((H_DIM, H_DIM), lambda i: (0, 0)),
        ],
        out_specs=pl.BlockSpec((128, H_DIM), lambda i: (i, 0)),
        out_shape=jax.ShapeDtypeStruct((N_TC, H_DIM), jnp.float32),
    )(p, h1p, dis, b1, w2)


def _tc_fuse3(p, h2p, dis, b2, wh_pad, bh_pad):
    return pl.pallas_call(
        _fuse3_body,
        grid=(N_BLK,),
        in_specs=[
            pl.BlockSpec((2, 128, H_DIM), lambda i: (0, i, 0)),
            pl.BlockSpec((128, H_DIM), lambda i: (i, 0)),
            pl.BlockSpec((128, 1), lambda i: (i, 0)),
            pl.BlockSpec((1, H_DIM), lambda i: (0, 0)),
            pl.BlockSpec((H_DIM, 128), lambda i: (0, 0)),
            pl.BlockSpec((1, 128), lambda i: (0, 0)),
        ],
        out_specs=pl.BlockSpec((128, 128), lambda i: (i, 0)),
        out_shape=jax.ShapeDtypeStruct((N_TC, 128), jnp.float32),
    )(p, h2p, dis, b2, wh_pad, bh_pad)


def kernel(x, edge_index, W1, b1, W2, b2, Wh, bh):
    src = edge_index[0].astype(jnp.int32)
    dst = edge_index[1].astype(jnp.int32)
    src = jnp.concatenate([src, jnp.zeros((E_PAD - N_EDGES,), jnp.int32)])
    dst = jnp.concatenate(
        [dst, jnp.full((E_PAD - N_EDGES,), N_NODES, jnp.int32)])
    src2d = src.reshape(IDX_ROWS, 128)
    dst2d = dst.reshape(IDX_ROWS, 128)

    zeros_w = jnp.zeros((SLICE, H_DIM), jnp.float32)
    zeros_1 = jnp.zeros((SLICE,), jnp.float32)
    ones_1 = jnp.ones((128,), jnp.float32)

    x_pad = jnp.pad(x, ((0, N_TC - N_NODES), (0, 0)))
    wh_pad = jnp.pad(Wh, ((0, 0), (0, 128 - Wh.shape[1])))
    bh_pad = jnp.pad(bh, (0, 128 - bh.shape[0])).reshape(1, 128)
    b1r = b1.reshape(1, H_DIM)
    b2r = b2.reshape(1, H_DIM)

    degp = _sc_degree(dst2d, zeros_1, ones_1)
    degp = degp[:, :N_TC].reshape(2, N_TC, 1)
    dis, h1p = _tc_fuse1(degp, x_pad, W1)

    p1 = _sc_aggregate(h1p, src2d, dst2d, zeros_w)
    h2p = _tc_fuse2(p1[:, :N_TC], h1p, dis, b1r, W2)

    p2 = _sc_aggregate(h2p, src2d, dst2d, zeros_w)
    logits_pad = _tc_fuse3(p2[:, :N_TC], h2p, dis, b2r, wh_pad, bh_pad)
    return logits_pad[:N_NODES, :Wh.shape[1]]


# speedup vs baseline: 25.1581x; 1.5943x over previous
import functools

import jax
import jax.numpy as jnp
from jax import lax
from jax.experimental import pallas as pl
from jax.experimental.pallas import tpu as pltpu
from jax.experimental.pallas import tpu_sc as plsc

N_NODES = 10000
N_EDGES = 320000
IN_DIM = 128
H_DIM = 32

N_TILES = 32
E_PAD = 327680
IDX_ROWS = E_PAD // 128
ROWS_PER_TILE = IDX_ROWS // N_TILES
CH = 4
NC = ROWS_PER_TILE // CH
N_PAD = 10240
SLICE = N_PAD // 16
BLK = 2048
N_BLK = N_PAD // BLK

_mesh = plsc.VectorSubcoreMesh(core_axis_name="c", subcore_axis_name="s")


@functools.partial(
    pl.kernel,
    out_type=jax.ShapeDtypeStruct((2, N_PAD), jnp.float32),
    mesh=_mesh,
    scratch_types=[
        pltpu.VMEM((CH, 128), jnp.int32),
        pltpu.VMEM((SLICE,), jnp.float32),
        pltpu.VMEM((128,), jnp.float32),
        pltpu.VMEM_SHARED((N_PAD,), jnp.float32),
        pltpu.SemaphoreType.DMA,
        pltpu.SemaphoreType.DMA,
    ],
)
def _sc_degree(dst_hbm, zeros_hbm, ones_hbm, out_hbm,
               dst_v, stage_v, ones_v, deg_sh, sem0, sem1):
    c = lax.axis_index("c")
    s = lax.axis_index("s")
    wid = c * 16 + s
    base = wid * ROWS_PER_TILE
    pltpu.sync_copy(zeros_hbm, stage_v)
    pltpu.sync_copy(stage_v, deg_sh.at[pl.ds(s * SLICE, SLICE)])
    pltpu.sync_copy(ones_hbm, ones_v)
    plsc.subcore_barrier()

    def body(i, carry):
        pltpu.sync_copy(dst_hbm.at[pl.ds(base + i * CH, CH)], dst_v)
        for j in range(CH):
            pltpu.sync_copy(ones_v, deg_sh.at[dst_v.at[j]], add=True)
        return carry

    lax.fori_loop(0, NC, body, 0)
    plsc.subcore_barrier()
    pltpu.sync_copy(deg_sh.at[pl.ds(s * SLICE, SLICE)],
                    out_hbm.at[c, pl.ds(s * SLICE, SLICE)])


@functools.partial(
    pl.kernel,
    out_type=jax.ShapeDtypeStruct((2, N_PAD, H_DIM), jnp.float32),
    mesh=_mesh,
    scratch_types=[
        pltpu.VMEM((2, CH, 128), jnp.int32),
        pltpu.VMEM((2, CH, 128), jnp.int32),
        pltpu.VMEM((2, CH * 128, H_DIM), jnp.float32),
        pltpu.VMEM((SLICE, H_DIM), jnp.float32),
        pltpu.VMEM_SHARED((N_PAD, H_DIM), jnp.float32),
        pltpu.SemaphoreType.DMA,
        pltpu.SemaphoreType.DMA,
        pltpu.SemaphoreType.DMA,
        pltpu.SemaphoreType.DMA,
    ],
    compiler_params=pltpu.CompilerParams(use_tc_tiling_on_sc=False),
)
def _sc_aggregate(h_hbm, src_hbm, dst_hbm, zeros_hbm, out_hbm,
                  src_v, dst_v, rows_v, zbuf, acc_sh,
                  gsem0, gsem1, ssem0, ssem1):
    c = lax.axis_index("c")
    s = lax.axis_index("s")
    wid = c * 16 + s
    base = wid * ROWS_PER_TILE
    pltpu.sync_copy(zeros_hbm, zbuf)
    pltpu.sync_copy(zbuf, acc_sh.at[pl.ds(s * SLICE, SLICE)])
    plsc.subcore_barrier()

    gsem = (gsem0, gsem1)
    ssem = (ssem0, ssem1)

    def load_idx(k, p):
        r0 = base + k * CH
        pltpu.sync_copy(src_hbm.at[pl.ds(r0, CH)], src_v.at[p])
        pltpu.sync_copy(dst_hbm.at[pl.ds(r0, CH)], dst_v.at[p])

    def fire_g(p):
        for j in range(CH):
            pltpu.async_copy(h_hbm.at[src_v.at[p, j]],
                             rows_v.at[p, pl.ds(j * 128, 128)], gsem[p])

    def drain_g(p):
        for j in range(CH):
            pltpu.make_async_copy(h_hbm.at[src_v.at[p, j]],
                                  rows_v.at[p, pl.ds(j * 128, 128)],
                                  gsem[p]).wait()

    def fire_s(p):
        for j in range(CH):
            pltpu.async_copy(rows_v.at[p, pl.ds(j * 128, 128)],
                             acc_sh.at[dst_v.at[p, j]], ssem[p], add=True)

    def drain_s(p):
        for j in range(CH):
            pltpu.make_async_copy(rows_v.at[p, pl.ds(j * 128, 128)],
                                  acc_sh.at[dst_v.at[p, j]], ssem[p]).wait()

    load_idx(0, 0)
    fire_g(0)
    load_idx(1, 1)
    fire_g(1)

    def body(m, carry):
        k = 2 * m
        drain_g(0)
        fire_s(0)
        drain_s(0)
        load_idx(k + 2, 0)
        fire_g(0)
        drain_g(1)
        fire_s(1)
        drain_s(1)
        load_idx(k + 3, 1)
        fire_g(1)
        return carry

    lax.fori_loop(0, (NC - 2) // 2, body, 0)
    drain_g(0)
    fire_s(0)
    drain_s(0)
    drain_g(1)
    fire_s(1)
    drain_s(1)
    plsc.subcore_barrier()
    pltpu.sync_copy(acc_sh.at[pl.ds(s * SLICE, SLICE)],
                    out_hbm.at[c, pl.ds(s * SLICE, SLICE)])


def _fuse1_body(degp_ref, x_ref, w1_ref, dis_ref, h_ref):
    deg = degp_ref[0] + degp_ref[1] + 1.0
    dis = lax.rsqrt(deg)
    dis_ref[...] = dis
    h = jnp.dot(x_ref[...], w1_ref[...], preferred_element_type=jnp.float32)
    h_ref[...] = h * dis


def _fuse2_body(p_ref, h1p_ref, dis_ref, b1_ref, w2_ref, out_ref):
    dis = dis_ref[...]
    acc = p_ref[0] + p_ref[1] + h1p_ref[...]
    h1 = jnp.maximum(acc * dis + b1_ref[...], 0.0)
    out_ref[...] = jnp.dot(h1, w2_ref[...], preferred_element_type=jnp.float32) * dis


def _fuse3_body(p_ref, h2p_ref, dis_ref, b2_ref, wh_ref, bh_ref, out_ref):
    acc = p_ref[0] + p_ref[1] + h2p_ref[...]
    h2 = jnp.maximum(acc * dis_ref[...] + b2_ref[...], 0.0)
    out_ref[...] = jnp.dot(h2, wh_ref[...], preferred_element_type=jnp.float32) + bh_ref[...]


def _tc_fuse1(degp, x, w1):
    return pl.pallas_call(
        _fuse1_body,
        grid=(N_BLK,),
        in_specs=[
            pl.BlockSpec((2, BLK, 1), lambda i: (0, i, 0)),
            pl.BlockSpec((BLK, IN_DIM), lambda i: (i, 0)),
            pl.BlockSpec((IN_DIM, H_DIM), lambda i: (0, 0)),
        ],
        out_specs=[
            pl.BlockSpec((BLK, 1), lambda i: (i, 0)),
            pl.BlockSpec((BLK, H_DIM), lambda i: (i, 0)),
        ],
        out_shape=[
            jax.ShapeDtypeStruct((N_PAD, 1), jnp.float32),
            jax.ShapeDtypeStruct((N_PAD, H_DIM), jnp.float32),
        ],
    )(degp, x, w1)


def _tc_fuse2(p, h1p, dis, b1, w2):
    return pl.pallas_call(
        _fuse2_body,
        grid=(N_BLK,),
        in_specs=[
            pl.BlockSpec((2, BLK, H_DIM), lambda i: (0, i, 0)),
            pl.BlockSpec((BLK, H_DIM), lambda i: (i, 0)),
            pl.BlockSpec((BLK, 1), lambda i: (i, 0)),
            pl.BlockSpec((1, H_DIM), lambda i: (0, 0)),
            pl.BlockSpec((H_DIM, H_DIM), lambda i: (0, 0)),
        ],
        out_specs=pl.BlockSpec((BLK, H_DIM), lambda i: (i, 0)),
        out_shape=jax.ShapeDtypeStruct((N_PAD, H_DIM), jnp.float32),
    )(p, h1p, dis, b1, w2)


def _tc_fuse3(p, h2p, dis, b2, wh_pad, bh_pad):
    return pl.pallas_call(
        _fuse3_body,
        grid=(N_BLK,),
        in_specs=[
            pl.BlockSpec((2, BLK, H_DIM), lambda i: (0, i, 0)),
            pl.BlockSpec((BLK, H_DIM), lambda i: (i, 0)),
            pl.BlockSpec((BLK, 1), lambda i: (i, 0)),
            pl.BlockSpec((1, H_DIM), lambda i: (0, 0)),
            pl.BlockSpec((H_DIM, 128), lambda i: (0, 0)),
            pl.BlockSpec((1, 128), lambda i: (0, 0)),
        ],
        out_specs=pl.BlockSpec((BLK, 128), lambda i: (i, 0)),
        out_shape=jax.ShapeDtypeStruct((N_PAD, 128), jnp.float32),
    )(p, h2p, dis, b2, wh_pad, bh_pad)


def kernel(x, edge_index, W1, b1, W2, b2, Wh, bh):
    src = edge_index[0].astype(jnp.int32)
    dst = edge_index[1].astype(jnp.int32)
    src = jnp.concatenate([src, jnp.zeros((E_PAD - N_EDGES,), jnp.int32)])
    dst = jnp.concatenate(
        [dst, jnp.full((E_PAD - N_EDGES,), N_NODES, jnp.int32)])
    src2d = src.reshape(IDX_ROWS, 128)
    dst2d = dst.reshape(IDX_ROWS, 128)

    zeros_w = jnp.zeros((SLICE, H_DIM), jnp.float32)
    zeros_1 = jnp.zeros((SLICE,), jnp.float32)
    ones_1 = jnp.ones((128,), jnp.float32)

    wh_pad = jnp.pad(Wh, ((0, 0), (0, 128 - Wh.shape[1])))
    bh_pad = jnp.pad(bh, (0, 128 - bh.shape[0])).reshape(1, 128)
    b1r = b1.reshape(1, H_DIM)
    b2r = b2.reshape(1, H_DIM)

    degp = _sc_degree(dst2d, zeros_1, ones_1)
    degp = degp.reshape(2, N_PAD, 1)
    dis, h1p = _tc_fuse1(degp, x, W1)

    p1 = _sc_aggregate(h1p, src2d, dst2d, zeros_w)
    h2p = _tc_fuse2(p1, h1p, dis, b1r, W2)

    p2 = _sc_aggregate(h2p, src2d, dst2d, zeros_w)
    logits_pad = _tc_fuse3(p2, h2p, dis, b2r, wh_pad, bh_pad)
    return logits_pad[:N_NODES, :Wh.shape[1]]


# speedup vs baseline: 25.6951x; 1.0213x over previous
import functools

import jax
import jax.numpy as jnp
from jax import lax
from jax.experimental import pallas as pl
from jax.experimental.pallas import tpu as pltpu
from jax.experimental.pallas import tpu_sc as plsc

N_NODES = 10000
N_EDGES = 320000
IN_DIM = 128
H_DIM = 32

N_TILES = 32
E_PAD = 327680
IDX_ROWS = E_PAD // 128
ROWS_PER_TILE = IDX_ROWS // N_TILES
CH = 4
NC = ROWS_PER_TILE // CH
N_PAD = 10240
SLICE = N_PAD // 16
BLK = 2048
N_BLK = N_PAD // BLK

_mesh = plsc.VectorSubcoreMesh(core_axis_name="c", subcore_axis_name="s")


@functools.partial(
    pl.kernel,
    out_type=jax.ShapeDtypeStruct((2, N_PAD), jnp.float32),
    mesh=_mesh,
    scratch_types=[
        pltpu.VMEM((ROWS_PER_TILE, 128), jnp.int32),
        pltpu.VMEM((SLICE,), jnp.float32),
        pltpu.VMEM((128,), jnp.float32),
        pltpu.VMEM_SHARED((N_PAD,), jnp.float32),
        pltpu.SemaphoreType.DMA,
        pltpu.SemaphoreType.DMA,
    ],
)
def _sc_degree(dst_hbm, zeros_hbm, ones_hbm, out_hbm,
               dst_v, stage_v, ones_v, deg_sh, sem0, sem1):
    c = lax.axis_index("c")
    s = lax.axis_index("s")
    wid = c * 16 + s
    base = wid * ROWS_PER_TILE
    pltpu.sync_copy(dst_hbm.at[pl.ds(base, ROWS_PER_TILE)], dst_v)
    pltpu.sync_copy(zeros_hbm, stage_v)
    pltpu.sync_copy(stage_v, deg_sh.at[pl.ds(s * SLICE, SLICE)])
    pltpu.sync_copy(ones_hbm, ones_v)
    plsc.subcore_barrier()

    sems = (sem0, sem1)

    def fire(k, p):
        for j in range(CH):
            pltpu.async_copy(ones_v, deg_sh.at[dst_v.at[k * CH + j]],
                             sems[p], add=True)

    def drain(k, p):
        for j in range(CH):
            pltpu.make_async_copy(ones_v, deg_sh.at[dst_v.at[k * CH + j]],
                                  sems[p]).wait()

    fire(0, 0)

    def body(m, carry):
        k = 2 * m
        fire(k + 1, 1)
        drain(k, 0)
        fire(k + 2, 0)
        drain(k + 1, 1)
        return carry

    lax.fori_loop(0, (NC - 2) // 2, body, 0)
    fire(NC - 1, 1)
    drain(NC - 2, 0)
    drain(NC - 1, 1)
    plsc.subcore_barrier()
    pltpu.sync_copy(deg_sh.at[pl.ds(s * SLICE, SLICE)],
                    out_hbm.at[c, pl.ds(s * SLICE, SLICE)])


@functools.partial(
    pl.kernel,
    out_type=jax.ShapeDtypeStruct((2, N_PAD, H_DIM), jnp.float32),
    mesh=_mesh,
    scratch_types=[
        pltpu.VMEM((ROWS_PER_TILE, 128), jnp.int32),
        pltpu.VMEM((ROWS_PER_TILE, 128), jnp.int32),
        pltpu.VMEM((3, CH * 128, H_DIM), jnp.float32),
        pltpu.VMEM((SLICE, H_DIM), jnp.float32),
        pltpu.VMEM_SHARED((N_PAD, H_DIM), jnp.float32),
        pltpu.SemaphoreType.DMA,
        pltpu.SemaphoreType.DMA,
        pltpu.SemaphoreType.DMA,
        pltpu.SemaphoreType.DMA,
        pltpu.SemaphoreType.DMA,
        pltpu.SemaphoreType.DMA,
    ],
    compiler_params=pltpu.CompilerParams(use_tc_tiling_on_sc=False),
)
def _sc_aggregate(h_hbm, src_hbm, dst_hbm, zeros_hbm, out_hbm,
                  src_v, dst_v, rows_v, zbuf, acc_sh,
                  gsem0, gsem1, gsem2, ssem0, ssem1, ssem2):
    c = lax.axis_index("c")
    s = lax.axis_index("s")
    wid = c * 16 + s
    base = wid * ROWS_PER_TILE
    pltpu.sync_copy(src_hbm.at[pl.ds(base, ROWS_PER_TILE)], src_v)
    pltpu.sync_copy(dst_hbm.at[pl.ds(base, ROWS_PER_TILE)], dst_v)
    pltpu.sync_copy(zeros_hbm, zbuf)
    pltpu.sync_copy(zbuf, acc_sh.at[pl.ds(s * SLICE, SLICE)])
    plsc.subcore_barrier()

    gsem = (gsem0, gsem1, gsem2)
    ssem = (ssem0, ssem1, ssem2)

    def fire_g(k, p):
        for j in range(CH):
            pltpu.async_copy(h_hbm.at[src_v.at[k * CH + j]],
                             rows_v.at[p, pl.ds(j * 128, 128)], gsem[p])

    def drain_g(k, p):
        for j in range(CH):
            pltpu.make_async_copy(h_hbm.at[src_v.at[k * CH + j]],
                                  rows_v.at[p, pl.ds(j * 128, 128)],
                                  gsem[p]).wait()

    def fire_s(k, p):
        for j in range(CH):
            pltpu.async_copy(rows_v.at[p, pl.ds(j * 128, 128)],
                             acc_sh.at[dst_v.at[k * CH + j]], ssem[p],
                             add=True)

    def drain_s(k, p):
        for j in range(CH):
            pltpu.make_async_copy(rows_v.at[p, pl.ds(j * 128, 128)],
                                  acc_sh.at[dst_v.at[k * CH + j]],
                                  ssem[p]).wait()

    fire_g(0, 0)
    fire_g(1, 1)
    drain_g(0, 0)
    fire_s(0, 0)
    fire_g(2, 2)
    drain_g(1, 1)
    fire_s(1, 1)
    drain_s(0, 0)
    fire_g(3, 0)
    drain_g(2, 2)
    fire_s(2, 2)
    drain_s(1, 1)
    fire_g(4, 1)

    def body(m, carry):
        for t in range(3):
            k = 3 * m + 3 + t
            p = t
            pf = (t + 2) % 3
            drain_g(k, p)
            fire_s(k, p)
            drain_s(k - 1, pf)
            fire_g(k + 2, pf)
        return carry

    lax.fori_loop(0, (NC - 5) // 3, body, 0)
    k = NC - 2
    drain_g(k, k % 3)
    fire_s(k, k % 3)
    drain_s(k - 1, (k - 1) % 3)
    k = NC - 1
    drain_g(k, k % 3)
    fire_s(k, k % 3)
    drain_s(k - 1, (k - 1) % 3)
    drain_s(k, k % 3)
    plsc.subcore_barrier()
    pltpu.sync_copy(acc_sh.at[pl.ds(s * SLICE, SLICE)],
                    out_hbm.at[c, pl.ds(s * SLICE, SLICE)])


def _fuse1_body(degp_ref, x_ref, w1_ref, dis_ref, h_ref):
    deg = degp_ref[0] + degp_ref[1] + 1.0
    dis = lax.rsqrt(deg)
    dis_ref[...] = dis
    h = jnp.dot(x_ref[...], w1_ref[...], preferred_element_type=jnp.float32)
    h_ref[...] = h * dis


def _fuse2_body(p_ref, h1p_ref, dis_ref, b1_ref, w2_ref, out_ref):
    dis = dis_ref[...]
    acc = p_ref[0] + p_ref[1] + h1p_ref[...]
    h1 = jnp.maximum(acc * dis + b1_ref[...], 0.0)
    out_ref[...] = jnp.dot(h1, w2_ref[...], preferred_element_type=jnp.float32) * dis


def _fuse3_body(p_ref, h2p_ref, dis_ref, b2_ref, wh_ref, bh_ref, out_ref):
    acc = p_ref[0] + p_ref[1] + h2p_ref[...]
    h2 = jnp.maximum(acc * dis_ref[...] + b2_ref[...], 0.0)
    out_ref[...] = jnp.dot(h2, wh_ref[...], preferred_element_type=jnp.float32) + bh_ref[...]


def _tc_fuse1(degp, x, w1):
    return pl.pallas_call(
        _fuse1_body,
        grid=(N_BLK,),
        in_specs=[
            pl.BlockSpec((2, BLK, 1), lambda i: (0, i, 0)),
            pl.BlockSpec((BLK, IN_DIM), lambda i: (i, 0)),
            pl.BlockSpec((IN_DIM, H_DIM), lambda i: (0, 0)),
        ],
        out_specs=[
            pl.BlockSpec((BLK, 1), lambda i: (i, 0)),
            pl.BlockSpec((BLK, H_DIM), lambda i: (i, 0)),
        ],
        out_shape=[
            jax.ShapeDtypeStruct((N_PAD, 1), jnp.float32),
            jax.ShapeDtypeStruct((N_PAD, H_DIM), jnp.float32),
        ],
    )(degp, x, w1)


def _tc_fuse2(p, h1p, dis, b1, w2):
    return pl.pallas_call(
        _fuse2_body,
        grid=(N_BLK,),
        in_specs=[
            pl.BlockSpec((2, BLK, H_DIM), lambda i: (0, i, 0)),
            pl.BlockSpec((BLK, H_DIM), lambda i: (i, 0)),
            pl.BlockSpec((BLK, 1), lambda i: (i, 0)),
            pl.BlockSpec((1, H_DIM), lambda i: (0, 0)),
            pl.BlockSpec((H_DIM, H_DIM), lambda i: (0, 0)),
        ],
        out_specs=pl.BlockSpec((BLK, H_DIM), lambda i: (i, 0)),
        out_shape=jax.ShapeDtypeStruct((N_PAD, H_DIM), jnp.float32),
    )(p, h1p, dis, b1, w2)


def _tc_fuse3(p, h2p, dis, b2, wh_pad, bh_pad):
    return pl.pallas_call(
        _fuse3_body,
        grid=(N_BLK,),
        in_specs=[
            pl.BlockSpec((2, BLK, H_DIM), lambda i: (0, i, 0)),
            pl.BlockSpec((BLK, H_DIM), lambda i: (i, 0)),
            pl.BlockSpec((BLK, 1), lambda i: (i, 0)),
            pl.BlockSpec((1, H_DIM), lambda i: (0, 0)),
            pl.BlockSpec((H_DIM, 128), lambda i: (0, 0)),
            pl.BlockSpec((1, 128), lambda i: (0, 0)),
        ],
        out_specs=pl.BlockSpec((BLK, 128), lambda i: (i, 0)),
        out_shape=jax.ShapeDtypeStruct((N_PAD, 128), jnp.float32),
    )(p, h2p, dis, b2, wh_pad, bh_pad)


def kernel(x, edge_index, W1, b1, W2, b2, Wh, bh):
    src = edge_index[0].astype(jnp.int32)
    dst = edge_index[1].astype(jnp.int32)
    src = jnp.concatenate([src, jnp.zeros((E_PAD - N_EDGES,), jnp.int32)])
    dst = jnp.concatenate(
        [dst, jnp.full((E_PAD - N_EDGES,), N_NODES, jnp.int32)])
    src2d = src.reshape(IDX_ROWS, 128)
    dst2d = dst.reshape(IDX_ROWS, 128)

    zeros_w = jnp.zeros((SLICE, H_DIM), jnp.float32)
    zeros_1 = jnp.zeros((SLICE,), jnp.float32)
    ones_1 = jnp.ones((128,), jnp.float32)

    wh_pad = jnp.pad(Wh, ((0, 0), (0, 128 - Wh.shape[1])))
    bh_pad = jnp.pad(bh, (0, 128 - bh.shape[0])).reshape(1, 128)
    b1r = b1.reshape(1, H_DIM)
    b2r = b2.reshape(1, H_DIM)

    degp = _sc_degree(dst2d, zeros_1, ones_1)
    degp = degp.reshape(2, N_PAD, 1)
    dis, h1p = _tc_fuse1(degp, x, W1)

    p1 = _sc_aggregate(h1p, src2d, dst2d, zeros_w)
    h2p = _tc_fuse2(p1, h1p, dis, b1r, W2)

    p2 = _sc_aggregate(h2p, src2d, dst2d, zeros_w)
    logits_pad = _tc_fuse3(p2, h2p, dis, b2r, wh_pad, bh_pad)
    return logits_pad[:N_NODES, :Wh.shape[1]]
